# baseline (speedup 1.0000x reference)
import jax
import jax.numpy as jnp
from jax.experimental import pallas as pl


def _tc_body(w_ref, x_ref, o_ref):
    o_ref[...] = jnp.sum(x_ref[...] * w_ref[...], axis=1, keepdims=True)


def kernel(items_emb, user_w):
    m, dim = items_emb.shape
    blk = 8000
    out = pl.pallas_call(
        _tc_body,
        grid=(m // blk,),
        in_specs=[
            pl.BlockSpec((1, dim), lambda i: (0, 0)),
            pl.BlockSpec((blk, dim), lambda i: (i, 0)),
        ],
        out_specs=pl.BlockSpec((blk, 1), lambda i: (i, 0)),
        out_shape=jax.ShapeDtypeStruct((m, 1), jnp.float32),
    )(user_w, items_emb)
    return out.reshape(m)


# speedup vs baseline: 10.1901x; 10.1901x over previous
import jax
import jax.numpy as jnp
from jax.experimental import pallas as pl


def _tc_body(w_ref, x_ref, o_ref):
    o_ref[...] = jnp.sum(x_ref[...] * w_ref[...], axis=0)


def kernel(items_emb, user_w):
    m, dim = items_emb.shape
    xt = items_emb.T
    w_col = user_w.reshape(dim, 1)
    blk = 32768
    grid = (m + blk - 1) // blk
    out = pl.pallas_call(
        _tc_body,
        grid=(grid,),
        in_specs=[
            pl.BlockSpec((dim, 1), lambda i: (0, 0)),
            pl.BlockSpec((dim, blk), lambda i: (0, i)),
        ],
        out_specs=pl.BlockSpec((blk,), lambda i: (i,)),
        out_shape=jax.ShapeDtypeStruct((m,), jnp.float32),
    )(w_col, xt)
    return out
